# baseline (device time: 60210 ns/iter reference)
import jax
import jax.numpy as jnp
from jax import lax
from jax.experimental import pallas as pl
from jax.experimental.pallas import tpu as pltpu

N_DEV = 16


def kernel(x, router_W, route_idx, expert_W):
    n_tok, d = x.shape
    e_local, _, h = expert_W.shape
    n_exp = router_W.shape[1]
    rows_per = n_tok // N_DEV

    def body(x_ref, rw_ref, idx_ref, ew_ref, out_ref,
             partial_ref, comm_ref, send_sems, recv_sems):
        my = lax.axis_index("i")
        left = lax.rem(my - 1 + N_DEV, N_DEV)
        right = lax.rem(my + 1, N_DEV)

        barrier_sem = pltpu.get_barrier_semaphore()
        for nbr in (left, right):
            pl.semaphore_signal(
                barrier_sem, inc=1,
                device_id=(nbr,), device_id_type=pl.DeviceIdType.MESH,
            )
        pl.semaphore_wait(barrier_sem, 2)

        xf = x_ref[:, :]
        scores = jnp.dot(xf, rw_ref[:, :], preferred_element_type=jnp.float32)
        scores = scores - jnp.max(scores, axis=-1, keepdims=True)
        probs = jnp.exp(scores)
        probs = probs / jnp.sum(probs, axis=-1, keepdims=True)

        idx0 = idx_ref[:, 0:1]
        idx1 = idx_ref[:, 1:2]
        eids = lax.broadcasted_iota(jnp.int32, (n_tok, n_exp), 1)
        onehot = (idx0 == eids) | (idx1 == eids)
        top2 = jnp.where(onehot, probs, 0.0)
        gates = top2 / jnp.sum(top2, axis=-1, keepdims=True)

        acc = jnp.zeros((n_tok, h), jnp.float32)
        for k in range(e_local):
            e_id = my * e_local + k
            gate_k = jnp.sum(
                jnp.where(eids == e_id, gates, 0.0), axis=-1, keepdims=True
            )
            xg = (xf * gate_k).astype(jnp.bfloat16)
            acc = acc + jnp.dot(
                xg, ew_ref[k, :, :].astype(jnp.bfloat16),
                preferred_element_type=jnp.float32,
            )
        partial_ref[:, :] = acc

        c0 = lax.rem(my - 1 + N_DEV, N_DEV)
        comm_ref[0, :, :] = partial_ref[pl.ds(c0 * rows_per, rows_per), :]

        for s in range(N_DEV - 1):
            rdma = pltpu.make_async_remote_copy(
                src_ref=comm_ref.at[s],
                dst_ref=comm_ref.at[s + 1],
                send_sem=send_sems.at[s],
                recv_sem=recv_sems.at[s],
                device_id=(right,),
                device_id_type=pl.DeviceIdType.MESH,
            )
            rdma.start()
            rdma.wait()

            c_recv = lax.rem(my - 2 - s + 2 * N_DEV, N_DEV)
            comm_ref[s + 1, :, :] = (
                comm_ref[s + 1, :, :]
                + partial_ref[pl.ds(c_recv * rows_per, rows_per), :]
            )

        out_ref[:, :] = comm_ref[N_DEV - 1, :, :]

    return pl.pallas_call(
        body,
        out_shape=jax.ShapeDtypeStruct((rows_per, h), jnp.float32),
        in_specs=[
            pl.BlockSpec(memory_space=pltpu.VMEM),
            pl.BlockSpec(memory_space=pltpu.VMEM),
            pl.BlockSpec(memory_space=pltpu.VMEM),
            pl.BlockSpec(memory_space=pltpu.VMEM),
        ],
        out_specs=pl.BlockSpec(memory_space=pltpu.VMEM),
        scratch_shapes=[
            pltpu.VMEM((n_tok, h), jnp.float32),
            pltpu.VMEM((N_DEV, rows_per, h), jnp.float32),
            pltpu.SemaphoreType.DMA((N_DEV - 1,)),
            pltpu.SemaphoreType.DMA((N_DEV - 1,)),
        ],
        compiler_params=pltpu.CompilerParams(collective_id=0),
    )(x, router_W, route_idx, expert_W)


# device time: 30401 ns/iter; 1.9805x vs baseline; 1.9805x over previous
import jax
import jax.numpy as jnp
from jax import lax
from jax.experimental import pallas as pl
from jax.experimental.pallas import tpu as pltpu

N_DEV = 16


def _perm(c: int) -> int:
    zc, qc = c // 4, c % 4
    return (qc // 2) * 8 + (qc % 2) * 4 + (zc % 2) * 2 + (zc // 2)


def kernel(x, router_W, route_idx, expert_W):
    n_tok, d = x.shape
    e_local, _, h = expert_W.shape
    n_exp = router_W.shape[1]
    rows_per = n_tok // N_DEV

    def body(x_ref, rw_ref, idx_ref, ew_ref, out_ref,
             partial_ref, stage_ref, send_sems, recv_sems):
        my = lax.axis_index("i")
        q = lax.rem(my, 4)
        zz = my // 4
        q1, q0 = q // 2, lax.rem(q, 2)
        z1, z0 = zz // 2, lax.rem(zz, 2)

        partners = (
            4 * zz + (3 - q),
            4 * zz + 2 * q1 + (1 - q0),
            my + 4 * (1 - 2 * z0),
            my + 8 * (1 - 2 * z1),
        )
        bits = (q1, q0, z0, z1)
        halves = (n_tok // 2, n_tok // 4, n_tok // 8, n_tok // 16)
        stage_off = (0, n_tok // 2, 3 * n_tok // 4, 7 * n_tok // 8)

        barrier_sem = pltpu.get_barrier_semaphore()
        for p in partners:
            pl.semaphore_signal(
                barrier_sem, inc=1,
                device_id=(p,), device_id_type=pl.DeviceIdType.MESH,
            )

        xf = x_ref[:, :]
        scores = jnp.dot(xf, rw_ref[:, :], preferred_element_type=jnp.float32)
        scores = scores - jnp.max(scores, axis=-1, keepdims=True)
        probs = jnp.exp(scores)
        probs = probs / jnp.sum(probs, axis=-1, keepdims=True)

        idx0 = idx_ref[:, 0:1]
        idx1 = idx_ref[:, 1:2]
        eids = lax.broadcasted_iota(jnp.int32, (n_tok, n_exp), 1)
        onehot = (idx0 == eids) | (idx1 == eids)
        top2 = jnp.where(onehot, probs, 0.0)
        gates = top2 / jnp.sum(top2, axis=-1, keepdims=True)

        acc = jnp.zeros((n_tok, h), jnp.float32)
        for k in range(e_local):
            e_id = my * e_local + k
            gate_k = jnp.sum(
                jnp.where(eids == e_id, gates, 0.0), axis=-1, keepdims=True
            )
            xg = (xf * gate_k).astype(jnp.bfloat16)
            acc = acc + jnp.dot(
                xg, ew_ref[k, :, :].astype(jnp.bfloat16),
                preferred_element_type=jnp.float32,
            )
        accb = acc.astype(jnp.bfloat16)

        for c in range(N_DEV):
            partial_ref[pl.ds(_perm(c) * rows_per, rows_per), :] = (
                accb[c * rows_per:(c + 1) * rows_per, :]
            )

        pl.semaphore_wait(barrier_sem, len(partners))

        base = 0
        for t in range(4):
            half = halves[t]
            keep = base + bits[t] * half
            send = base + (1 - bits[t]) * half
            rdma = pltpu.make_async_remote_copy(
                src_ref=partial_ref.at[pl.ds(send, half)],
                dst_ref=stage_ref.at[pl.ds(stage_off[t], half)],
                send_sem=send_sems.at[t],
                recv_sem=recv_sems.at[t],
                device_id=(partners[t],),
                device_id_type=pl.DeviceIdType.MESH,
            )
            rdma.start()
            rdma.wait()
            partial_ref[pl.ds(keep, half), :] = (
                partial_ref[pl.ds(keep, half), :]
                + stage_ref[pl.ds(stage_off[t], half), :]
            )
            base = keep

        out_ref[:, :] = partial_ref[pl.ds(base, rows_per), :].astype(jnp.float32)

    return pl.pallas_call(
        body,
        out_shape=jax.ShapeDtypeStruct((rows_per, h), jnp.float32),
        in_specs=[
            pl.BlockSpec(memory_space=pltpu.VMEM),
            pl.BlockSpec(memory_space=pltpu.VMEM),
            pl.BlockSpec(memory_space=pltpu.VMEM),
            pl.BlockSpec(memory_space=pltpu.VMEM),
        ],
        out_specs=pl.BlockSpec(memory_space=pltpu.VMEM),
        scratch_shapes=[
            pltpu.VMEM((n_tok, h), jnp.bfloat16),
            pltpu.VMEM((15 * n_tok // 16, h), jnp.bfloat16),
            pltpu.SemaphoreType.DMA((4,)),
            pltpu.SemaphoreType.DMA((4,)),
        ],
        compiler_params=pltpu.CompilerParams(collective_id=0),
    )(x, router_W, route_idx, expert_W)


# device time: 24693 ns/iter; 2.4383x vs baseline; 1.2312x over previous
import jax
import jax.numpy as jnp
from jax import lax
from jax.experimental import pallas as pl
from jax.experimental.pallas import tpu as pltpu

N_DEV = 16


def kernel(x, router_W, route_idx, expert_W):
    n_tok, d = x.shape
    e_local, _, h = expert_W.shape
    n_exp = router_W.shape[1]
    rows_per = n_tok // N_DEV
    qrows = 4 * rows_per

    def body(x_ref, rw_ref, idx_ref, ew_ref, out_ref,
             partial_ref, stage_ref, send_sems, recv_sems):
        my = lax.axis_index("i")
        q = lax.rem(my, 4)
        zz = my // 4

        barrier_sem = pltpu.get_barrier_semaphore()
        for t in range(1, 4):
            pl.semaphore_signal(
                barrier_sem, inc=1,
                device_id=(4 * zz + jnp.bitwise_xor(q, t),),
                device_id_type=pl.DeviceIdType.MESH,
            )
            pl.semaphore_signal(
                barrier_sem, inc=1,
                device_id=(4 * jnp.bitwise_xor(zz, t) + q,),
                device_id_type=pl.DeviceIdType.MESH,
            )

        xf = x_ref[:, :]
        xb = xf.astype(jnp.bfloat16)
        scores = jnp.dot(
            xb, rw_ref[:, :].astype(jnp.bfloat16),
            preferred_element_type=jnp.float32,
        )
        scores = scores - jnp.max(scores, axis=-1, keepdims=True)
        probs = jnp.exp(scores)
        probs = probs / jnp.sum(probs, axis=-1, keepdims=True)

        idx0 = idx_ref[:, 0:1]
        idx1 = idx_ref[:, 1:2]
        eids = lax.broadcasted_iota(jnp.int32, (n_tok, n_exp), 1)
        onehot = (idx0 == eids) | (idx1 == eids)
        top2 = jnp.where(onehot, probs, 0.0)
        gates = top2 / jnp.sum(top2, axis=-1, keepdims=True)

        acc = jnp.zeros((n_tok, h), jnp.float32)
        for k in range(e_local):
            e_id = my * e_local + k
            gate_k = jnp.sum(
                jnp.where(eids == e_id, gates, 0.0), axis=-1, keepdims=True
            )
            xg = (xf * gate_k).astype(jnp.bfloat16)
            acc = acc + jnp.dot(
                xg, ew_ref[k, :, :].astype(jnp.bfloat16),
                preferred_element_type=jnp.float32,
            )
        accb = acc.astype(jnp.bfloat16)

        for c in range(N_DEV):
            pos = (c % 4) * 4 + c // 4
            partial_ref[pl.ds(pos * rows_per, rows_per), :] = (
                accb[c * rows_per:(c + 1) * rows_per, :]
            )

        pl.semaphore_wait(barrier_sem, 6)

        s1 = []
        for t in range(1, 4):
            mate_q = jnp.bitwise_xor(q, t)
            rdma = pltpu.make_async_remote_copy(
                src_ref=partial_ref.at[pl.ds(mate_q * qrows, qrows)],
                dst_ref=stage_ref.at[pl.ds((t - 1) * qrows, qrows)],
                send_sem=send_sems.at[t - 1],
                recv_sem=recv_sems.at[t - 1],
                device_id=(4 * zz + mate_q,),
                device_id_type=pl.DeviceIdType.MESH,
            )
            rdma.start()
            s1.append(rdma)
        for rdma in s1:
            rdma.wait()
        partial_ref[pl.ds(q * qrows, qrows), :] = (
            partial_ref[pl.ds(q * qrows, qrows), :]
            + stage_ref[pl.ds(0, qrows), :]
            + stage_ref[pl.ds(qrows, qrows), :]
            + stage_ref[pl.ds(2 * qrows, qrows), :]
        )

        s2 = []
        for t in range(1, 4):
            mate_z = jnp.bitwise_xor(zz, t)
            rdma = pltpu.make_async_remote_copy(
                src_ref=partial_ref.at[pl.ds(q * qrows + mate_z * rows_per,
                                             rows_per)],
                dst_ref=stage_ref.at[pl.ds(3 * qrows + (t - 1) * rows_per,
                                           rows_per)],
                send_sem=send_sems.at[3 + t - 1],
                recv_sem=recv_sems.at[3 + t - 1],
                device_id=(4 * mate_z + q,),
                device_id_type=pl.DeviceIdType.MESH,
            )
            rdma.start()
            s2.append(rdma)
        for rdma in s2:
            rdma.wait()

        out_ref[:, :] = (
            partial_ref[pl.ds(q * qrows + zz * rows_per, rows_per), :]
            + stage_ref[pl.ds(3 * qrows, rows_per), :]
            + stage_ref[pl.ds(3 * qrows + rows_per, rows_per), :]
            + stage_ref[pl.ds(3 * qrows + 2 * rows_per, rows_per), :]
        ).astype(jnp.float32)

    return pl.pallas_call(
        body,
        out_shape=jax.ShapeDtypeStruct((rows_per, h), jnp.float32),
        in_specs=[
            pl.BlockSpec(memory_space=pltpu.VMEM),
            pl.BlockSpec(memory_space=pltpu.VMEM),
            pl.BlockSpec(memory_space=pltpu.VMEM),
            pl.BlockSpec(memory_space=pltpu.VMEM),
        ],
        out_specs=pl.BlockSpec(memory_space=pltpu.VMEM),
        scratch_shapes=[
            pltpu.VMEM((n_tok, h), jnp.bfloat16),
            pltpu.VMEM((3 * qrows + 3 * rows_per, h), jnp.bfloat16),
            pltpu.SemaphoreType.DMA((6,)),
            pltpu.SemaphoreType.DMA((6,)),
        ],
        compiler_params=pltpu.CompilerParams(collective_id=0),
    )(x, router_W, route_idx, expert_W)
